# baseline (speedup 1.0000x reference)
import functools

import jax
import jax.numpy as jnp
from jax import lax
from jax.experimental import pallas as pl
from jax.experimental.pallas import tpu as pltpu
from jax.experimental.pallas import tpu_sc as plsc

_HI = lax.Precision.HIGHEST

_LP = (0.99999981055573, -0.49997450516904496, 0.33276187400767593,
       -0.2449965663963085, 0.17757117522338742, -0.10785469067556722,
       0.0442147247476005, -0.008574780333609729)

_NSUB = 16
_NSC = 2
_LANES = 16
_CW = 16


def _embed_body(x_ref, w_ref, b_ref, o_ref):
    z = jnp.dot(x_ref[...], w_ref[...],
                preferred_element_type=jnp.float32)
    o_ref[...] = jnp.maximum(z + b_ref[...], 0.0)


def _tables_body(h_ref, w_ref, td_ref, ts_ref):
    z = jnp.dot(h_ref[...], w_ref[...],
                preferred_element_type=jnp.float32)
    d = td_ref.shape[1]
    td_ref[...] = z[:, :d]
    ts_ref[...] = z[:, d:]


def _etab_body(ea_ref, w_ref, b_ref, o_ref):
    z = jnp.dot(ea_ref[...], w_ref[...],
                preferred_element_type=jnp.float32)
    o_ref[...] = z + b_ref[...]


def _count_body(dr_ref, dc_ref, o_ref):
    i = pl.program_id(0)
    nhi = o_ref.shape[0]
    hi_ids = lax.broadcasted_iota(jnp.int32, (nhi, 1), 0)
    oht = (lax.shift_right_logical(dr_ref[...], 7) == hi_ids)
    lo_ids = lax.broadcasted_iota(jnp.int32, (1, 128), 1)
    ol = (lax.bitwise_and(dc_ref[...], 127) == lo_ids)
    part = jnp.dot(oht.astype(jnp.float32), ol.astype(jnp.float32),
                   preferred_element_type=jnp.float32)

    @pl.when(i == 0)
    def _():
        o_ref[...] = part

    @pl.when(i > 0)
    def _():
        o_ref[...] += part


def _node_body(p_ref, cnt_ref, h_ref, g_ref, be_ref, o_ref):
    d1 = h_ref.shape[1]
    n = h_ref.shape[0]
    p = p_ref[0, :n, :] + p_ref[1, :n, :]
    agg = p / jnp.maximum(cnt_ref[...], 1.0)
    mu = jnp.mean(agg, axis=0, keepdims=True)
    dev = agg - mu
    var = jnp.mean(dev * dev, axis=0, keepdims=True)
    bn = g_ref[...] * dev / jnp.sqrt(var + 1e-5) + be_ref[...]
    o_ref[...] = jnp.maximum(bn + h_ref[...], 0.0)


def _pool_body(o_ref, b_ref, w1_ref, b1_ref, wa_ref, ba_ref, wb_ref, bb_ref,
               w2_ref, b2_ref, y_ref):
    g = w1_ref.shape[1]
    gcol = lax.broadcasted_iota(jnp.int32, (g, 1), 0)
    onehot = (b_ref[...] == gcol).astype(jnp.float32)
    pooled = jnp.dot(onehot, o_ref[...], precision=_HI,
                     preferred_element_type=jnp.float32)
    cnt = jnp.sum(onehot, axis=1, keepdims=True)
    h = pooled / jnp.maximum(cnt, 1.0)
    h = jnp.maximum(jnp.dot(h, w1_ref[...], ) + b1_ref[...], 0.)
    h = jnp.maximum(jnp.dot(h, wa_ref[...], ) + ba_ref[...], 0.)
    h = jnp.maximum(jnp.dot(h, wb_ref[...], ) + bb_ref[...], 0.)
    y_ref[...] = jnp.dot(h, w2_ref[...], ) + b2_ref[...]


def _edge_body(n_pad, n_edges, d1, blk,
               td_hbm, ts_hbm, ef_hbm, dst_hbm, src_hbm, z_hbm, out_hbm,
               gdx0, gdx1, gsx0, gsx1, sdx0, sdx1,
               td0, td1, ts0, ts1, ef0, ef1, m0, m1, acc,
               smi0, smi1, smg0, smg1, sms0, sms1, smsi0, smsi1):
    cid = lax.axis_index("c")
    sid = lax.axis_index("s")
    wid = cid * _NSUB + sid
    rows = n_pad // _NSUB
    pltpu.sync_copy(z_hbm, acc.at[pl.ds(sid * rows, rows)])

    e_per_tile = n_edges // (_NSC * _NSUB)
    base0 = wid * e_per_tile
    nit = e_per_tile // blk

    gdx = (gdx0, gdx1)
    gsx = (gsx0, gsx1)
    sdx = (sdx0, sdx1)
    tdb = (td0, td1)
    tsb = (ts0, ts1)
    efb = (ef0, ef1)
    mb = (m0, m1)
    smi = (smi0, smi1)
    smg = (smg0, smg1)
    sms = (sms0, sms1)
    smsi = (smsi0, smsi1)

    def issue_idx(i, p):
        b = base0 + i * blk
        pltpu.async_copy(dst_hbm.at[pl.ds(b, blk)], gdx[p], smi[p])
        pltpu.async_copy(src_hbm.at[pl.ds(b, blk)], gsx[p], smi[p])

    def wait_idx(p):
        pltpu.make_async_copy(dst_hbm.at[pl.ds(0, blk)], gdx[p], smi[p]).wait()
        pltpu.make_async_copy(src_hbm.at[pl.ds(0, blk)], gsx[p], smi[p]).wait()

    def issue_gather(i, p):
        b = base0 + i * blk
        pltpu.async_copy(ef_hbm.at[pl.ds(b, blk)], efb[p], smg[p])
        pltpu.async_copy(td_hbm.at[gdx[p]], tdb[p], smg[p])
        pltpu.async_copy(ts_hbm.at[gsx[p]], tsb[p], smg[p])

    def wait_gather(p):
        pltpu.make_async_copy(ef_hbm.at[pl.ds(0, blk)], efb[p], smg[p]).wait()
        pltpu.make_async_copy(td_hbm.at[gdx[p]], tdb[p], smg[p]).wait()
        pltpu.make_async_copy(ts_hbm.at[gsx[p]], tsb[p], smg[p]).wait()

    def compute(p):
        td_b, ts_b, ef_b, m_b = tdb[p], tsb[p], efb[p], mb[p]

        @plsc.parallel_loop(0, blk, unroll=4)
        def _(e):
            for k in range(d1 // _LANES):
                o = _LANES * k
                nf = (td_b[e, pl.ds(o, _LANES)] + ts_b[e, pl.ds(o, _LANES)]
                      + ef_b[e, pl.ds(o, _LANES)])
                sg = 1.0 / (1.0 + jnp.exp(nf))
                s = (td_b[e, pl.ds(d1 + o, _LANES)]
                     + ts_b[e, pl.ds(d1 + o, _LANES)]
                     + ef_b[e, pl.ds(d1 + o, _LANES)])
                u = jnp.exp(-jnp.abs(s))
                q = jnp.full((_LANES,), _LP[-1], jnp.float32)
                for c in _LP[-2::-1]:
                    q = q * u + c
                sp = jnp.maximum(s, 0.0) + u * q
                m_b[e, pl.ds(o, _LANES)] = sg * sp

    def step(i, p, q):
        wait_gather(p)

        @pl.when(i + 2 < nit)
        def _():
            issue_idx(i + 2, p)

        @pl.when(i + 1 < nit)
        def _():
            wait_idx(q)
            issue_gather(i + 1, q)

        @pl.when(i >= 2)
        def _():
            pltpu.make_async_copy(mb[p], acc.at[sdx[p]], sms[p]).wait()

        pltpu.async_copy(dst_hbm.at[pl.ds(base0 + i * blk, blk)],
                         sdx[p], smsi[p])
        compute(p)
        pltpu.make_async_copy(dst_hbm.at[pl.ds(0, blk)], sdx[p],
                              smsi[p]).wait()
        pltpu.async_copy(mb[p], acc.at[sdx[p]], sms[p], add=True)

    issue_idx(0, 0)
    issue_idx(1, 1)
    wait_idx(0)
    issue_gather(0, 0)

    @pl.loop(0, nit)
    def _(i):
        @pl.when(i % 2 == 0)
        def _():
            step(i, 0, 1)

        @pl.when(i % 2 == 1)
        def _():
            step(i, 1, 0)

    pltpu.make_async_copy(mb[0], acc.at[sdx[0]], sms[0]).wait()
    pltpu.make_async_copy(mb[1], acc.at[sdx[1]], sms[1]).wait()

    plsc.subcore_barrier()
    pltpu.sync_copy(acc.at[pl.ds(sid * rows, rows)],
                    out_hbm.at[cid, pl.ds(sid * rows, rows)])


@functools.lru_cache(maxsize=None)
def _make_edge_kernel(n_pad, n_edges, d1, blk):
    mesh = plsc.VectorSubcoreMesh(core_axis_name="c", subcore_axis_name="s")
    body = functools.partial(_edge_body, n_pad, n_edges, d1, blk)
    return pl.kernel(
        body,
        out_type=jax.ShapeDtypeStruct((_NSC, n_pad, d1), jnp.float32),
        mesh=mesh,
        scratch_types=(
            [pltpu.VMEM((blk,), jnp.int32)] * 6
            + [pltpu.VMEM((blk, 2 * d1), jnp.float32)] * 6
            + [pltpu.VMEM((blk, d1), jnp.float32)] * 2
            + [pltpu.VMEM_SHARED((n_pad, d1), jnp.float32)]
            + [pltpu.SemaphoreType.DMA] * 8
        ),
    )


def kernel(x, edge_index, edge_attr, batch, params):
    n, feat = x.shape
    e = edge_index.shape[1]
    d1 = params["W0"].shape[1]
    de = edge_attr.shape[1]
    g = 64
    blk = 16

    f32 = jnp.float32
    n_pad = ((n + 128 - 1) // 128) * 128
    dst = edge_index[1]
    src = edge_index[0]
    zrows = jnp.zeros((n_pad // _NSUB, d1), f32)

    ce_blk = 16000
    cnt2d = pl.pallas_call(
        _count_body,
        grid=(e // ce_blk,),
        in_specs=[pl.BlockSpec((1, ce_blk), lambda i: (0, i)),
                  pl.BlockSpec((ce_blk, 1), lambda i: (i, 0))],
        out_specs=pl.BlockSpec((n_pad // 128, 128), lambda i: (0, 0)),
        out_shape=jax.ShapeDtypeStruct((n_pad // 128, 128), f32),
    )(dst.reshape(1, e), dst.reshape(e, 1))
    cnt = cnt2d.reshape(-1)[:n].reshape(n, 1)

    h = pl.pallas_call(
        _embed_body,
        out_shape=jax.ShapeDtypeStruct((n, d1), f32),
    )(x, params["W0"], params["b0"].reshape(1, d1))

    edge_kernel = _make_edge_kernel(n_pad, e, d1, blk)

    be_blk = 4000
    for c in params["convs"]:
        wf, ws = c["Wf"], c["Ws"]
        wn = jnp.concatenate(
            [-wf[:d1], ws[:d1], -wf[d1:2 * d1], ws[d1:2 * d1]], axis=1)
        td, ts = pl.pallas_call(
            _tables_body,
            grid=(5,),
            in_specs=[pl.BlockSpec((n // 5, d1), lambda i: (i, 0)),
                      pl.BlockSpec((d1, 4 * d1), lambda i: (0, 0))],
            out_specs=[pl.BlockSpec((n // 5, 2 * d1), lambda i: (i, 0)),
                       pl.BlockSpec((n // 5, 2 * d1), lambda i: (i, 0))],
            out_shape=[jax.ShapeDtypeStruct((n, 2 * d1), f32),
                       jax.ShapeDtypeStruct((n, 2 * d1), f32)],
        )(h, wn)
        we = jnp.concatenate([-wf[2 * d1:], ws[2 * d1:]], axis=1)
        bbe = jnp.concatenate([-c["bf"], c["bs"]]).reshape(1, 2 * d1)
        ef = pl.pallas_call(
            _etab_body,
            grid=(e // be_blk,),
            in_specs=[pl.BlockSpec((be_blk, de), lambda i: (i, 0)),
                      pl.BlockSpec((de, 2 * d1), lambda i: (0, 0)),
                      pl.BlockSpec((1, 2 * d1), lambda i: (0, 0))],
            out_specs=pl.BlockSpec((be_blk, 2 * d1), lambda i: (i, 0)),
            out_shape=jax.ShapeDtypeStruct((e, 2 * d1), f32),
        )(edge_attr, we, bbe)

        parts = edge_kernel(td, ts, ef, dst, src, zrows)

        h = pl.pallas_call(
            _node_body,
            out_shape=jax.ShapeDtypeStruct((n, d1), f32),
        )(parts, cnt, h, c["gamma"].reshape(1, d1), c["beta"].reshape(1, d1))

    (wfa, bfa), (wfb, bfb) = params["fc"]
    y = pl.pallas_call(
        _pool_body,
        out_shape=jax.ShapeDtypeStruct((g, 1), f32),
    )(h, batch.reshape(1, n), params["W1"], params["b1"].reshape(1, -1),
      wfa, bfa.reshape(1, -1), wfb, bfb.reshape(1, -1),
      params["W2"], params["b2"].reshape(1, 1))
    return y.reshape(-1)


# speedup vs baseline: 2.3815x; 2.3815x over previous
import functools

import jax
import jax.numpy as jnp
from jax import lax
from jax.experimental import pallas as pl
from jax.experimental.pallas import tpu as pltpu
from jax.experimental.pallas import tpu_sc as plsc

_HI = lax.Precision.HIGHEST

_LP = (0.99999981055573, -0.49997450516904496, 0.33276187400767593,
       -0.2449965663963085, 0.17757117522338742, -0.10785469067556722,
       0.0442147247476005, -0.008574780333609729)

_NSUB = 16
_NSC = 2
_LANES = 16
_CW = 16


def _embed_body(x_ref, w_ref, b_ref, o_ref):
    z = jnp.dot(x_ref[...], w_ref[...],
                preferred_element_type=jnp.float32)
    o_ref[...] = jnp.maximum(z + b_ref[...], 0.0)


def _tables_body(h_ref, w_ref, td_ref, ts_ref):
    z = jnp.dot(h_ref[...], w_ref[...],
                preferred_element_type=jnp.float32)
    d = td_ref.shape[1]
    td_ref[...] = z[:, :d]
    ts_ref[...] = z[:, d:]


def _etab_body(ea_ref, w_ref, b_ref, o_ref):
    z = jnp.dot(ea_ref[...], w_ref[...],
                preferred_element_type=jnp.float32)
    o_ref[...] = z + b_ref[...]


def _count_body(dr_ref, dc_ref, o_ref):
    i = pl.program_id(0)
    nhi = o_ref.shape[0]
    hi_ids = lax.broadcasted_iota(jnp.int32, (nhi, 1), 0)
    oht = (lax.shift_right_logical(dr_ref[...], 7) == hi_ids)
    lo_ids = lax.broadcasted_iota(jnp.int32, (1, 128), 1)
    ol = (lax.bitwise_and(dc_ref[...], 127) == lo_ids)
    part = jnp.dot(oht.astype(jnp.float32), ol.astype(jnp.float32),
                   preferred_element_type=jnp.float32)

    @pl.when(i == 0)
    def _():
        o_ref[...] = part

    @pl.when(i > 0)
    def _():
        o_ref[...] += part


def _node_body(p_ref, cnt_ref, h_ref, g_ref, be_ref, o_ref):
    d1 = h_ref.shape[1]
    n = h_ref.shape[0]
    p = p_ref[0, :n, :] + p_ref[1, :n, :]
    agg = p / jnp.maximum(cnt_ref[...], 1.0)
    mu = jnp.mean(agg, axis=0, keepdims=True)
    dev = agg - mu
    var = jnp.mean(dev * dev, axis=0, keepdims=True)
    bn = g_ref[...] * dev / jnp.sqrt(var + 1e-5) + be_ref[...]
    o_ref[...] = jnp.maximum(bn + h_ref[...], 0.0)


def _pool_body(o_ref, b_ref, w1_ref, b1_ref, wa_ref, ba_ref, wb_ref, bb_ref,
               w2_ref, b2_ref, y_ref):
    g = w1_ref.shape[1]
    gcol = lax.broadcasted_iota(jnp.int32, (g, 1), 0)
    onehot = (b_ref[...] == gcol).astype(jnp.float32)
    pooled = jnp.dot(onehot, o_ref[...], precision=_HI,
                     preferred_element_type=jnp.float32)
    cnt = jnp.sum(onehot, axis=1, keepdims=True)
    h = pooled / jnp.maximum(cnt, 1.0)
    h = jnp.maximum(jnp.dot(h, w1_ref[...], ) + b1_ref[...], 0.)
    h = jnp.maximum(jnp.dot(h, wa_ref[...], ) + ba_ref[...], 0.)
    h = jnp.maximum(jnp.dot(h, wb_ref[...], ) + bb_ref[...], 0.)
    y_ref[...] = jnp.dot(h, w2_ref[...], ) + b2_ref[...]


def _edge_body(n_pad, n_edges, d1, blk,
               td_hbm, ts_hbm, ef_hbm, dst_hbm, src_hbm, z_hbm, out_hbm,
               gdx0, gdx1, gsx0, gsx1, sdx0, sdx1,
               td0, td1, ts0, ts1, ef0, ef1, m0, m1, acc,
               smi0, smi1, smg0, smg1, sms0, sms1, smsi0, smsi1):
    cid = lax.axis_index("c")
    sid = lax.axis_index("s")
    wid = cid * _NSUB + sid
    rows = n_pad // _NSUB
    pltpu.sync_copy(z_hbm, acc.at[pl.ds(sid * rows, rows)])

    e_per_tile = n_edges // (_NSC * _NSUB)
    base0 = wid * e_per_tile
    nit = e_per_tile // blk

    gdx = (gdx0, gdx1)
    gsx = (gsx0, gsx1)
    sdx = (sdx0, sdx1)
    tdb = (td0, td1)
    tsb = (ts0, ts1)
    efb = (ef0, ef1)
    mb = (m0, m1)
    smi = (smi0, smi1)
    smg = (smg0, smg1)
    sms = (sms0, sms1)
    smsi = (smsi0, smsi1)

    def issue_idx(i, p):
        b = base0 + i * blk
        pltpu.async_copy(dst_hbm.at[pl.ds(b, blk)], gdx[p], smi[p])
        pltpu.async_copy(src_hbm.at[pl.ds(b, blk)], gsx[p], smi[p])

    def wait_idx(p):
        pltpu.make_async_copy(dst_hbm.at[pl.ds(0, blk)], gdx[p], smi[p]).wait()
        pltpu.make_async_copy(src_hbm.at[pl.ds(0, blk)], gsx[p], smi[p]).wait()

    def issue_gather(i, p):
        b = base0 + i * blk
        pltpu.async_copy(ef_hbm.at[pl.ds(b, blk)], efb[p], smg[p])
        pltpu.async_copy(td_hbm.at[gdx[p]], tdb[p], smg[p])
        pltpu.async_copy(ts_hbm.at[gsx[p]], tsb[p], smg[p])

    def wait_gather(p):
        pltpu.make_async_copy(ef_hbm.at[pl.ds(0, blk)], efb[p], smg[p]).wait()
        pltpu.make_async_copy(td_hbm.at[gdx[p]], tdb[p], smg[p]).wait()
        pltpu.make_async_copy(ts_hbm.at[gsx[p]], tsb[p], smg[p]).wait()

    def compute(p):
        td_b, ts_b, ef_b, m_b = tdb[p], tsb[p], efb[p], mb[p]

        @plsc.parallel_loop(0, blk, unroll=2)
        def _(e):
            for k in range(d1 // _LANES):
                o = _LANES * k
                nf = (td_b[e, pl.ds(o, _LANES)] + ts_b[e, pl.ds(o, _LANES)]
                      + ef_b[e, pl.ds(o, _LANES)])
                sg = 1.0 / (1.0 + jnp.exp(nf))
                s = (td_b[e, pl.ds(d1 + o, _LANES)]
                     + ts_b[e, pl.ds(d1 + o, _LANES)]
                     + ef_b[e, pl.ds(d1 + o, _LANES)])
                u = jnp.exp(-jnp.abs(s))
                q = jnp.full((_LANES,), _LP[-1], jnp.float32)
                for c in _LP[-2::-1]:
                    q = q * u + c
                sp = jnp.maximum(s, 0.0) + u * q
                m_b[e, pl.ds(o, _LANES)] = sg * sp

    def step(i, p, q):
        wait_gather(p)

        @pl.when(i + 2 < nit)
        def _():
            issue_idx(i + 2, p)

        @pl.when(i + 1 < nit)
        def _():
            wait_idx(q)
            issue_gather(i + 1, q)

        @pl.when(i >= 2)
        def _():
            pltpu.make_async_copy(mb[p], acc.at[sdx[p]], sms[p]).wait()

        pltpu.async_copy(dst_hbm.at[pl.ds(base0 + i * blk, blk)],
                         sdx[p], smsi[p])
        compute(p)
        pltpu.make_async_copy(dst_hbm.at[pl.ds(0, blk)], sdx[p],
                              smsi[p]).wait()
        pltpu.async_copy(mb[p], acc.at[sdx[p]], sms[p], add=True)

    issue_idx(0, 0)
    issue_idx(1, 1)
    wait_idx(0)
    issue_gather(0, 0)

    @pl.loop(0, nit)
    def _(i):
        @pl.when(i % 2 == 0)
        def _():
            step(i, 0, 1)

        @pl.when(i % 2 == 1)
        def _():
            step(i, 1, 0)

    pltpu.make_async_copy(mb[0], acc.at[sdx[0]], sms[0]).wait()
    pltpu.make_async_copy(mb[1], acc.at[sdx[1]], sms[1]).wait()

    plsc.subcore_barrier()
    pltpu.sync_copy(acc.at[pl.ds(sid * rows, rows)],
                    out_hbm.at[cid, pl.ds(sid * rows, rows)])


@functools.lru_cache(maxsize=None)
def _make_edge_kernel(n_pad, n_edges, d1, blk):
    mesh = plsc.VectorSubcoreMesh(core_axis_name="c", subcore_axis_name="s")
    body = functools.partial(_edge_body, n_pad, n_edges, d1, blk)
    return pl.kernel(
        body,
        out_type=jax.ShapeDtypeStruct((_NSC, n_pad, d1), jnp.float32),
        mesh=mesh,
        scratch_types=(
            [pltpu.VMEM((blk,), jnp.int32)] * 6
            + [pltpu.VMEM((blk, 2 * d1), jnp.float32)] * 6
            + [pltpu.VMEM((blk, d1), jnp.float32)] * 2
            + [pltpu.VMEM_SHARED((n_pad, d1), jnp.float32)]
            + [pltpu.SemaphoreType.DMA] * 8
        ),
    )


def kernel(x, edge_index, edge_attr, batch, params):
    n, feat = x.shape
    e = edge_index.shape[1]
    d1 = params["W0"].shape[1]
    de = edge_attr.shape[1]
    g = 64
    blk = 16

    f32 = jnp.float32
    n_pad = ((n + 128 - 1) // 128) * 128
    dst = edge_index[1]
    src = edge_index[0]
    zrows = jnp.zeros((n_pad // _NSUB, d1), f32)

    ce_blk = 16000
    cnt2d = pl.pallas_call(
        _count_body,
        grid=(e // ce_blk,),
        in_specs=[pl.BlockSpec((1, ce_blk), lambda i: (0, i)),
                  pl.BlockSpec((ce_blk, 1), lambda i: (i, 0))],
        out_specs=pl.BlockSpec((n_pad // 128, 128), lambda i: (0, 0)),
        out_shape=jax.ShapeDtypeStruct((n_pad // 128, 128), f32),
    )(dst.reshape(1, e), dst.reshape(e, 1))
    cnt = cnt2d.reshape(-1)[:n].reshape(n, 1)

    h = pl.pallas_call(
        _embed_body,
        out_shape=jax.ShapeDtypeStruct((n, d1), f32),
    )(x, params["W0"], params["b0"].reshape(1, d1))

    edge_kernel = _make_edge_kernel(n_pad, e, d1, blk)

    be_blk = 4000
    for c in params["convs"]:
        wf, ws = c["Wf"], c["Ws"]
        wn = jnp.concatenate(
            [-wf[:d1], ws[:d1], -wf[d1:2 * d1], ws[d1:2 * d1]], axis=1)
        td, ts = pl.pallas_call(
            _tables_body,
            grid=(5,),
            in_specs=[pl.BlockSpec((n // 5, d1), lambda i: (i, 0)),
                      pl.BlockSpec((d1, 4 * d1), lambda i: (0, 0))],
            out_specs=[pl.BlockSpec((n // 5, 2 * d1), lambda i: (i, 0)),
                       pl.BlockSpec((n // 5, 2 * d1), lambda i: (i, 0))],
            out_shape=[jax.ShapeDtypeStruct((n, 2 * d1), f32),
                       jax.ShapeDtypeStruct((n, 2 * d1), f32)],
        )(h, wn)
        we = jnp.concatenate([-wf[2 * d1:], ws[2 * d1:]], axis=1)
        bbe = jnp.concatenate([-c["bf"], c["bs"]]).reshape(1, 2 * d1)
        ef = pl.pallas_call(
            _etab_body,
            grid=(e // be_blk,),
            in_specs=[pl.BlockSpec((be_blk, de), lambda i: (i, 0)),
                      pl.BlockSpec((de, 2 * d1), lambda i: (0, 0)),
                      pl.BlockSpec((1, 2 * d1), lambda i: (0, 0))],
            out_specs=pl.BlockSpec((be_blk, 2 * d1), lambda i: (i, 0)),
            out_shape=jax.ShapeDtypeStruct((e, 2 * d1), f32),
        )(edge_attr, we, bbe)

        parts = edge_kernel(td, ts, ef, dst, src, zrows)

        h = pl.pallas_call(
            _node_body,
            out_shape=jax.ShapeDtypeStruct((n, d1), f32),
        )(parts, cnt, h, c["gamma"].reshape(1, d1), c["beta"].reshape(1, d1))

    (wfa, bfa), (wfb, bfb) = params["fc"]
    y = pl.pallas_call(
        _pool_body,
        out_shape=jax.ShapeDtypeStruct((g, 1), f32),
    )(h, batch.reshape(1, n), params["W1"], params["b1"].reshape(1, -1),
      wfa, bfa.reshape(1, -1), wfb, bfb.reshape(1, -1),
      params["W2"], params["b2"].reshape(1, 1))
    return y.reshape(-1)
